# baseline (speedup 1.0000x reference)
import functools

import jax
import jax.numpy as jnp
from jax import lax
from jax.experimental import pallas as pl
from jax.experimental.pallas import tpu as pltpu
from jax.experimental.pallas import tpu_sc as plsc

_TC_R = 256
_W = 8


def _sc_gather(table, index_h):
    N, C = table.shape
    BH = index_h.shape[0]
    info = plsc.get_sparse_core_info()
    nw = info.num_cores * info.num_subcores
    L = info.num_lanes
    b_per_w = BH // nw
    nwaves = b_per_w // L
    mesh = plsc.VectorSubcoreMesh(core_axis_name="c", subcore_axis_name="s")

    @functools.partial(
        pl.kernel,
        mesh=mesh,
        out_type=jax.ShapeDtypeStruct((BH, C), jnp.float32),
        scratch_types=[
            pltpu.VMEM((b_per_w,), jnp.int32),
            pltpu.VMEM((b_per_w, C), jnp.float32),
            pltpu.SemaphoreType.DMA,
        ],
    )
    def gather(table_hbm, idx_hbm, out_hbm, idx_v, rows_v, sem):
        cid = lax.axis_index("c")
        sid = lax.axis_index("s")
        wid = sid * info.num_cores + cid
        base = wid * b_per_w
        pltpu.sync_copy(idx_hbm.at[pl.ds(base, b_per_w)], idx_v)

        def issue_wave(w):
            chunk = idx_v[pl.ds(w * L, L)]
            for j in range(L):
                row = chunk[j]
                pltpu.make_async_copy(
                    table_hbm.at[pl.ds(row, 1), :],
                    rows_v.at[pl.ds(w * L + j, 1), :],
                    sem,
                ).start()

        def drain_wave(w):
            pltpu.make_async_copy(
                table_hbm.at[pl.ds(0, L), :],
                rows_v.at[pl.ds(w * L, L), :],
                sem,
            ).wait()

        issue_wave(0)
        for w in range(1, nwaves):
            issue_wave(w)
            drain_wave(w - 1)
        drain_wave(nwaves - 1)
        pltpu.sync_copy(rows_v, out_hbm.at[pl.ds(base, b_per_w)])

    return gather(table, index_h)


def _tc_body(x_ref, g_ref, loss_ref, acc_ref):
    i = pl.program_id(0)

    x = x_ref[...]
    g = g_ref[...]
    m = jnp.max(x, axis=1, keepdims=True)
    lse = m + jnp.log(jnp.sum(jnp.exp(x - m), axis=1, keepdims=True))
    s1 = jnp.sum(g, axis=1, keepdims=True)
    d = jnp.sum(x * g, axis=1, keepdims=True)
    loss = lse * s1 - d
    loss_ref[...] = loss

    @pl.when(i == 0)
    def _():
        acc_ref[...] = jnp.zeros_like(acc_ref)

    acc_ref[...] += jnp.sum(loss).reshape(1, 1)


def _tc_half(outputs, gathered, half):
    B, C = outputs.shape
    BH = gathered.shape[0]
    G = BH // _TC_R
    return pl.pallas_call(
        _tc_body,
        grid=(G,),
        in_specs=[
            pl.BlockSpec((_TC_R, C), lambda i: (i + half * G, 0)),
            pl.BlockSpec((_TC_R, C), lambda i: (i, 0)),
        ],
        out_specs=[
            pl.BlockSpec((_TC_R, 1), lambda i: (i, 0)),
            pl.BlockSpec((1, 1), lambda i: (0, 0)),
        ],
        out_shape=[
            jax.ShapeDtypeStruct((BH, 1), jnp.float32),
            jax.ShapeDtypeStruct((1, 1), jnp.float32),
        ],
    )(outputs, gathered)


def kernel(outputs, index, confidence):
    B, C = outputs.shape
    BH = B // 2
    ga = _sc_gather(confidence, index[:BH])
    gb = _sc_gather(confidence, index[BH:])
    loss_a, sum_a = _tc_half(outputs, ga, 0)
    loss_b, sum_b = _tc_half(outputs, gb, 1)
    avg = (sum_a[0, 0] + sum_b[0, 0]) / B
    loss_vec = jnp.concatenate([loss_a, loss_b], axis=0).reshape(B)
    return (avg, loss_vec)


# speedup vs baseline: 1.0147x; 1.0147x over previous
import functools

import jax
import jax.numpy as jnp
from jax import lax
from jax.experimental import pallas as pl
from jax.experimental.pallas import tpu as pltpu
from jax.experimental.pallas import tpu_sc as plsc

_TC_R = 256
_SC_BPW = 120


def _sc_gather(table, index_h):
    N, C = table.shape
    BH = index_h.shape[0]
    info = plsc.get_sparse_core_info()
    nw = info.num_cores * info.num_subcores
    L = info.num_lanes
    b_per_w = BH // nw
    waves = [L] * (b_per_w // L)
    if b_per_w % L:
        waves.append(b_per_w % L)
    mesh = plsc.VectorSubcoreMesh(core_axis_name="c", subcore_axis_name="s")

    @functools.partial(
        pl.kernel,
        mesh=mesh,
        out_type=jax.ShapeDtypeStruct((BH, C), jnp.float32),
        scratch_types=[
            pltpu.VMEM((len(waves) * L,), jnp.int32),
            pltpu.VMEM((b_per_w, C), jnp.float32),
            pltpu.SemaphoreType.DMA,
        ],
    )
    def gather(table_hbm, idx_hbm, out_hbm, idx_v, rows_v, sem):
        cid = lax.axis_index("c")
        sid = lax.axis_index("s")
        wid = sid * info.num_cores + cid
        base = wid * b_per_w
        pltpu.sync_copy(
            idx_hbm.at[pl.ds(base, b_per_w)], idx_v.at[pl.ds(0, b_per_w)]
        )

        def issue_wave(w, n):
            chunk = idx_v[pl.ds(w * L, L)]
            for j in range(n):
                row = chunk[j]
                pltpu.make_async_copy(
                    table_hbm.at[pl.ds(row, 1), :],
                    rows_v.at[pl.ds(w * L + j, 1), :],
                    sem,
                ).start()

        def drain_wave(w, n):
            pltpu.make_async_copy(
                table_hbm.at[pl.ds(0, n), :],
                rows_v.at[pl.ds(w * L, n), :],
                sem,
            ).wait()

        issue_wave(0, waves[0])
        for w in range(1, len(waves)):
            issue_wave(w, waves[w])
            drain_wave(w - 1, waves[w - 1])
        drain_wave(len(waves) - 1, waves[-1])
        pltpu.sync_copy(rows_v, out_hbm.at[pl.ds(base, b_per_w)])

    return gather(table, index_h)


def _loss_part(x, g):
    m = jnp.max(x, axis=1, keepdims=True)
    lse = m + jnp.log(jnp.sum(jnp.exp(x - m), axis=1, keepdims=True))
    s1 = jnp.sum(g, axis=1, keepdims=True)
    d = jnp.sum(x * g, axis=1, keepdims=True)
    return lse * s1 - d


def _tc_body(idx_ref, x_ref, g_ref, conf_hbm, loss_ref, acc_ref, buf, sem):
    i = pl.program_id(0)
    nsteps = pl.num_programs(0)

    @pl.when(i == 0)
    def _():
        acc_ref[...] = jnp.zeros_like(acc_ref)

        def issue_one(k, carry):
            row = idx_ref[(nsteps - 1) * _TC_R + k]
            pltpu.make_async_copy(
                conf_hbm.at[pl.ds(row, 1), :],
                buf.at[pl.ds(k, 1), :],
                sem,
            ).start()
            return carry

        jax.lax.fori_loop(0, _TC_R, issue_one, 0, unroll=8)

    x = x_ref[...]

    @pl.when(i < nsteps - 1)
    def _():
        loss = _loss_part(x, g_ref[...])
        loss_ref[...] = loss
        acc_ref[...] += jnp.sum(loss).reshape(1, 1)

    @pl.when(i == nsteps - 1)
    def _():
        pltpu.make_async_copy(conf_hbm.at[pl.ds(0, _TC_R), :], buf, sem).wait()
        loss = _loss_part(x, buf[...])
        loss_ref[...] = loss
        total = acc_ref[...] + jnp.sum(loss).reshape(1, 1)
        acc_ref[...] = total / (nsteps * _TC_R)


def kernel(outputs, index, confidence):
    B, C = outputs.shape
    nw = 32
    BH = _SC_BPW * nw
    G = B // _TC_R
    gathered = _sc_gather(confidence, index[:BH])
    GH = BH // _TC_R

    grid_spec = pltpu.PrefetchScalarGridSpec(
        num_scalar_prefetch=1,
        grid=(G,),
        in_specs=[
            pl.BlockSpec((_TC_R, C), lambda i, idx: (i, 0)),
            pl.BlockSpec((_TC_R, C), lambda i, idx: (jnp.minimum(i, GH - 1), 0)),
            pl.BlockSpec(memory_space=pl.ANY),
        ],
        out_specs=[
            pl.BlockSpec((_TC_R, 1), lambda i, idx: (i, 0)),
            pl.BlockSpec((1, 1), lambda i, idx: (0, 0)),
        ],
        scratch_shapes=[
            pltpu.VMEM((_TC_R, C), jnp.float32),
            pltpu.SemaphoreType.DMA,
        ],
    )
    loss2, acc = pl.pallas_call(
        _tc_body,
        grid_spec=grid_spec,
        out_shape=[
            jax.ShapeDtypeStruct((B, 1), jnp.float32),
            jax.ShapeDtypeStruct((1, 1), jnp.float32),
        ],
    )(index, outputs, gathered, confidence)
    return (acc[0, 0], loss2.reshape(B))


# speedup vs baseline: 1.0173x; 1.0026x over previous
import functools

import jax
import jax.numpy as jnp
from jax import lax
from jax.experimental import pallas as pl
from jax.experimental.pallas import tpu as pltpu
from jax.experimental.pallas import tpu_sc as plsc

_TC_R = 256
_SC_BPW = 120


def _sc_gather(table, index_h):
    N, C = table.shape
    BH = index_h.shape[0]
    info = plsc.get_sparse_core_info()
    nw = info.num_cores * info.num_subcores
    L = info.num_lanes
    b_per_w = BH // nw
    waves = [L] * (b_per_w // L)
    if b_per_w % L:
        waves.append(b_per_w % L)
    mesh = plsc.VectorSubcoreMesh(core_axis_name="c", subcore_axis_name="s")

    @functools.partial(
        pl.kernel,
        mesh=mesh,
        out_type=jax.ShapeDtypeStruct((BH, C), jnp.float32),
        scratch_types=[
            pltpu.VMEM((len(waves) * L,), jnp.int32),
            pltpu.VMEM((b_per_w, C), jnp.float32),
            pltpu.SemaphoreType.DMA,
        ],
        compiler_params=pltpu.CompilerParams(use_tc_tiling_on_sc=True),
    )
    def gather(table_hbm, idx_hbm, out_hbm, idx_v, rows_v, sem):
        cid = lax.axis_index("c")
        sid = lax.axis_index("s")
        wid = sid * info.num_cores + cid
        base = wid * b_per_w
        pltpu.sync_copy(
            idx_hbm.at[pl.ds(base, b_per_w)], idx_v.at[pl.ds(0, b_per_w)]
        )

        def issue_wave(w, n):
            chunk = idx_v[pl.ds(w * L, L)]
            for j in range(n):
                row = chunk[j]
                pltpu.make_async_copy(
                    table_hbm.at[pl.ds(row, 1), :],
                    rows_v.at[pl.ds(w * L + j, 1), :],
                    sem,
                ).start()

        def drain_wave(w, n):
            pltpu.make_async_copy(
                table_hbm.at[pl.ds(0, n), :],
                rows_v.at[pl.ds(w * L, n), :],
                sem,
            ).wait()

        issue_wave(0, waves[0])
        for w in range(1, len(waves)):
            issue_wave(w, waves[w])
            drain_wave(w - 1, waves[w - 1])
        drain_wave(len(waves) - 1, waves[-1])
        pltpu.sync_copy(rows_v, out_hbm.at[pl.ds(base, b_per_w)])

    return gather(table, index_h)


def _loss_part(x, g):
    m = jnp.max(x, axis=1, keepdims=True)
    lse = m + jnp.log(jnp.sum(jnp.exp(x - m), axis=1, keepdims=True))
    s1 = jnp.sum(g, axis=1, keepdims=True)
    d = jnp.sum(x * g, axis=1, keepdims=True)
    return lse * s1 - d


def _tc_body(idx_ref, x_ref, g_ref, conf_hbm, loss_ref, acc_ref, buf, sem):
    i = pl.program_id(0)
    nsteps = pl.num_programs(0)

    @pl.when(i == 0)
    def _():
        acc_ref[...] = jnp.zeros_like(acc_ref)

        def issue_one(k, carry):
            row = idx_ref[(nsteps - 1) * _TC_R + k]
            pltpu.make_async_copy(
                conf_hbm.at[pl.ds(row, 1), :],
                buf.at[pl.ds(k, 1), :],
                sem,
            ).start()
            return carry

        jax.lax.fori_loop(0, _TC_R, issue_one, 0, unroll=8)

    x = x_ref[...]

    @pl.when(i < nsteps - 1)
    def _():
        loss = _loss_part(x, g_ref[...])
        loss_ref[...] = loss
        acc_ref[...] += jnp.sum(loss).reshape(1, 1)

    @pl.when(i == nsteps - 1)
    def _():
        pltpu.make_async_copy(conf_hbm.at[pl.ds(0, _TC_R), :], buf, sem).wait()
        loss = _loss_part(x, buf[...])
        loss_ref[...] = loss
        total = acc_ref[...] + jnp.sum(loss).reshape(1, 1)
        acc_ref[...] = total / (nsteps * _TC_R)


def kernel(outputs, index, confidence):
    B, C = outputs.shape
    nw = 32
    BH = _SC_BPW * nw
    G = B // _TC_R
    gathered = _sc_gather(confidence, index[:BH])
    GH = BH // _TC_R

    grid_spec = pltpu.PrefetchScalarGridSpec(
        num_scalar_prefetch=1,
        grid=(G,),
        in_specs=[
            pl.BlockSpec((_TC_R, C), lambda i, idx: (i, 0)),
            pl.BlockSpec((_TC_R, C), lambda i, idx: (jnp.minimum(i, GH - 1), 0)),
            pl.BlockSpec(memory_space=pl.ANY),
        ],
        out_specs=[
            pl.BlockSpec((_TC_R, 1), lambda i, idx: (i, 0)),
            pl.BlockSpec((1, 1), lambda i, idx: (0, 0)),
        ],
        scratch_shapes=[
            pltpu.VMEM((_TC_R, C), jnp.float32),
            pltpu.SemaphoreType.DMA,
        ],
    )
    loss2, acc = pl.pallas_call(
        _tc_body,
        grid_spec=grid_spec,
        out_shape=[
            jax.ShapeDtypeStruct((B, 1), jnp.float32),
            jax.ShapeDtypeStruct((1, 1), jnp.float32),
        ],
    )(index, outputs, gathered, confidence)
    return (acc[0, 0], loss2.reshape(B))


# speedup vs baseline: 1.0560x; 1.0380x over previous
import jax
import jax.numpy as jnp
from jax.experimental import pallas as pl
from jax.experimental.pallas import tpu as pltpu

_R = 1024


def _issue_block(idx_ref, conf_hbm, buf, sem, step):
    base = step * _R

    def issue_one(k, carry):
        row = idx_ref[base + k]
        pltpu.make_async_copy(
            conf_hbm.at[pl.ds(row, 1), :],
            buf.at[pl.ds(k, 1), :],
            sem,
        ).start()
        return carry

    jax.lax.fori_loop(0, _R, issue_one, 0, unroll=8)


def _wait_block(conf_hbm, buf, sem):
    pltpu.make_async_copy(conf_hbm.at[pl.ds(0, _R), :], buf, sem).wait()


def _body(idx_ref, x_ref, conf_hbm, loss_ref, acc_ref, buf, sem):
    i = pl.program_id(0)
    nsteps = pl.num_programs(0)
    par = jax.lax.rem(i, 2)
    nxt = jax.lax.rem(i + 1, 2)

    @pl.when(i == 0)
    def _():
        _issue_block(idx_ref, conf_hbm, buf.at[0], sem.at[0], 0)

    @pl.when(i + 1 < nsteps)
    def _():
        _issue_block(idx_ref, conf_hbm, buf.at[nxt], sem.at[nxt], i + 1)

    _wait_block(conf_hbm, buf.at[par], sem.at[par])

    x = x_ref[...]
    g = buf[par]
    m = jnp.max(x, axis=1, keepdims=True)
    lse = m + jnp.log(jnp.sum(jnp.exp(x - m), axis=1, keepdims=True))
    s1 = jnp.sum(g, axis=1, keepdims=True)
    d = jnp.sum(x * g, axis=1, keepdims=True)
    loss = lse * s1 - d
    loss_ref[...] = loss

    @pl.when(i == 0)
    def _():
        acc_ref[...] = jnp.zeros_like(acc_ref)

    total = acc_ref[...] + jnp.sum(loss).reshape(1, 1)
    acc_ref[...] = total

    @pl.when(i == nsteps - 1)
    def _():
        acc_ref[...] = total / (nsteps * _R)


def kernel(outputs, index, confidence):
    B, C = outputs.shape
    G = B // _R
    grid_spec = pltpu.PrefetchScalarGridSpec(
        num_scalar_prefetch=1,
        grid=(G,),
        in_specs=[
            pl.BlockSpec((_R, C), lambda i, idx: (i, 0)),
            pl.BlockSpec(memory_space=pl.ANY),
        ],
        out_specs=[
            pl.BlockSpec((_R, 1), lambda i, idx: (i, 0)),
            pl.BlockSpec((1, 1), lambda i, idx: (0, 0)),
        ],
        scratch_shapes=[
            pltpu.VMEM((2, _R, C), jnp.float32),
            pltpu.SemaphoreType.DMA((2,)),
        ],
    )
    loss2, acc = pl.pallas_call(
        _body,
        grid_spec=grid_spec,
        out_shape=[
            jax.ShapeDtypeStruct((B, 1), jnp.float32),
            jax.ShapeDtypeStruct((1, 1), jnp.float32),
        ],
    )(index, outputs, confidence)
    return (acc[0, 0], loss2.reshape(B))
